# baseline (speedup 1.0000x reference)
import jax
import jax.numpy as jnp
from jax.experimental import pallas as pl
from jax.experimental.pallas import tpu as pltpu

_B, _R, _C = 4, 4096, 2048
_RBLK = 256
_NR = _R // _RBLK


def _argmin_body(x_ref, o_ref, m_ref, i_ref):
    r = pl.program_id(1)
    v = x_ref[0]
    bm = jnp.min(v, axis=0, keepdims=True)
    iota = jax.lax.broadcasted_iota(jnp.int32, v.shape, 0)
    bidx = jnp.min(jnp.where(v <= bm, iota, _RBLK), axis=0, keepdims=True)
    bidx = bidx + r * _RBLK

    @pl.when(r == 0)
    def _init():
        m_ref[...] = bm
        i_ref[...] = bidx

    @pl.when(r > 0)
    def _merge():
        take = bm < m_ref[...]
        i_ref[...] = jnp.where(take, bidx, i_ref[...])
        m_ref[...] = jnp.where(take, bm, m_ref[...])

    @pl.when(r == _NR - 1)
    def _emit():
        o_ref[0] = i_ref[...]


def kernel(x):
    out = pl.pallas_call(
        _argmin_body,
        grid=(_B, _NR),
        in_specs=[pl.BlockSpec((1, _RBLK, _C), lambda b, r: (b, r, 0))],
        out_specs=pl.BlockSpec((1, 1, _C), lambda b, r: (b, 0, 0)),
        out_shape=jax.ShapeDtypeStruct((_B, 1, _C), jnp.int32),
        scratch_shapes=[
            pltpu.VMEM((1, _C), jnp.float32),
            pltpu.VMEM((1, _C), jnp.int32),
        ],
        compiler_params=pltpu.CompilerParams(
            dimension_semantics=("parallel", "arbitrary"),
        ),
    )(x)
    return out.reshape(_B, _C).astype(jnp.int64)


# speedup vs baseline: 1.5850x; 1.5850x over previous
import jax
import jax.numpy as jnp
from jax.experimental import pallas as pl
from jax.experimental.pallas import tpu as pltpu

_B, _R, _C = 4, 4096, 2048
_CBLK = 1024


def _argmin_body(x_ref, o_ref):
    v = x_ref[0]
    m = jnp.min(v, axis=0, keepdims=True)
    iota = jax.lax.broadcasted_iota(jnp.int32, v.shape, 0)
    idx = jnp.min(jnp.where(v <= m, iota, _R), axis=0)
    o_ref[0, 0] = idx


def kernel(x):
    out = pl.pallas_call(
        _argmin_body,
        grid=(_B, _C // _CBLK),
        in_specs=[pl.BlockSpec((1, _R, _CBLK), lambda b, c: (b, 0, c))],
        out_specs=pl.BlockSpec((1, 1, _CBLK), lambda b, c: (b, 0, c)),
        out_shape=jax.ShapeDtypeStruct((_B, 1, _C), jnp.int32),
        compiler_params=pltpu.CompilerParams(
            dimension_semantics=("parallel", "parallel"),
        ),
    )(x)
    return out.reshape(_B, _C).astype(jnp.int64)
